# baseline (speedup 1.0000x reference)
import functools

import jax
import jax.numpy as jnp
from jax import lax
from jax.experimental import pallas as pl
from jax.experimental.pallas import tpu as pltpu
from jax.experimental.pallas import tpu_sc as plsc

N = 50000
E = 800000
H = 64
DEPTH = 6

NC = 2
NS = 16
CH = 128
HALF = N // NC

NCH = 392
G = 14
NG = NCH // G
EP_TILE = NCH * CH
EPAD = NS * EP_TILE

RACC = 25088
ZR = RACC // NS
ZCH = 112
TRASH_P = 25024

NCH_D = 392
DPAD = NC * NS * NCH_D * CH
DEG_R = 51200
DZR = DEG_R // NS
TRASH_D = 50176
DEG_W = 16

BLK = 1000
NBLK = N // BLK


def _make_prop():
    mesh = plsc.VectorSubcoreMesh(core_axis_name="c", subcore_axis_name="s")

    @functools.partial(
        pl.kernel,
        mesh=mesh,
        out_type=jax.ShapeDtypeStruct((NC, RACC, H), jnp.float32),
        scratch_types=[
            pltpu.VMEM((G, CH), jnp.int32),
            pltpu.VMEM((G, CH), jnp.int32),
            pltpu.VMEM((256,), jnp.int32),
            pltpu.VMEM((256,), jnp.int32),
            pltpu.VMEM((3, CH), jnp.int32),
            pltpu.VMEM((3, CH), jnp.int32),
            pltpu.VMEM((3, CH, H), jnp.float32),
            pltpu.VMEM_SHARED((RACC, H), jnp.float32),
            pltpu.SemaphoreType.DMA,
            pltpu.SemaphoreType.DMA,
            pltpu.SemaphoreType.DMA,
        ],
        compiler_params=pltpu.CompilerParams(use_tc_tiling_on_sc=False,
                                             needs_layout_passes=False),
    )
    def prop(hs_hbm, row_hbm, col_hbm, zb_hbm, out_hbm, rowi, coli, wr, wc,
             fr, fc, msg, acc, gs0, gs1, ssem):
        cid = lax.axis_index("c")
        sid = lax.axis_index("s")
        lo = cid * HALF
        zbase = sid * ZR

        def zloop(j, _):
            pltpu.async_copy(zb_hbm, acc.at[pl.ds(zbase + j * ZCH, ZCH)],
                             ssem)
            return 0

        lax.fori_loop(0, ZR // ZCH, zloop, 0)

        def zdrain(j, _):
            pltpu.make_async_copy(zb_hbm, acc.at[pl.ds(zbase + j * ZCH, ZCH)],
                                  ssem).wait()
            return 0

        lax.fori_loop(0, ZR // ZCH, zdrain, 0)
        plsc.subcore_barrier()

        def _drain_msg(b, sem):
            pltpu.make_async_copy(zb_hbm, msg.at[b, pl.ds(0, ZCH)],
                                  sem).wait()
            pltpu.make_async_copy(zb_hbm, msg.at[b, pl.ds(ZCH, CH - ZCH)],
                                  sem).wait()

        def _gdrain(fpar, b):
            def even():
                _drain_msg(b, gs0)
                return 0

            def odd():
                _drain_msg(b, gs1)
                return 0

            lax.cond(fpar % 2 == 0, even, odd)

        def _gissue(fpar, fb):
            def even():
                pltpu.async_copy(hs_hbm.at[fr.at[fb]], msg.at[fb], gs0)
                return 0

            def odd():
                pltpu.async_copy(hs_hbm.at[fr.at[fb]], msg.at[fb], gs1)
                return 0

            lax.cond(fpar % 2 == 0, even, odd)

        def fire(off, f):
            fb = f % 3
            for m in range(8):
                sl = pl.ds(16 * m, 16)
                fr[fb, sl] = wr[sl]
                fc[fb, sl] = wc[sl]
            for m in range(8):
                fr_sl = pl.ds(128 + 16 * m, 16)
                to_sl = pl.ds(16 * m, 16)
                wr[to_sl] = wr[fr_sl]
                wc[to_sl] = wc[fr_sl]

            @pl.when(f >= 3)
            def _():
                _drain_msg((f - 3) % 3, ssem)

            @pl.when(f >= 2)
            def _():
                qb = (f - 2) % 3
                _gdrain(f, qb)
                pltpu.async_copy(msg.at[qb], acc.at[fc.at[qb]], ssem,
                                 add=True)

            _gissue(f, fb)
            return off - 128, f + 1

        def gloop(g, carry):
            off, f = carry
            pltpu.sync_copy(row_hbm.at[sid, pl.ds(g * G, G)], rowi)
            pltpu.sync_copy(col_hbm.at[sid, pl.ds(g * G, G)], coli)
            for j in range(G):
                for k in range(8):
                    sl = pl.ds(16 * k, 16)
                    rv = rowi[j, sl]
                    cv = coli[j, sl]
                    msk = (cv >= lo) & (cv < lo + HALF)
                    loc = cv - lo
                    plsc.store_compressed(wr.at[pl.ds(off, 16)], rv, mask=msk)
                    plsc.store_compressed(wc.at[pl.ds(off, 16)], loc, mask=msk)
                    off = off + jnp.sum(jnp.where(msk, 1, 0))
                off, f = lax.cond(off >= 128, fire, lambda o, ff: (o, ff),
                                  off, f)
            return off, f

        off, f = lax.fori_loop(0, NG, gloop, (jnp.int32(0), jnp.int32(0)))

        iota16 = lax.iota(jnp.int32, 16)
        for m in range(8):
            sl = pl.ds(16 * m, 16)
            pos = 16 * m + iota16
            tail = pos >= off
            wr[sl] = jnp.where(tail, 0, wr[sl])
            wc[sl] = jnp.where(tail, TRASH_P + (pos & 63), wc[sl])
        off, f = fire(off, f)

        @pl.when(f >= 3)
        def _():
            _drain_msg((f - 3) % 3, ssem)

        @pl.when(f >= 2)
        def _():
            qb = (f - 2) % 3
            _gdrain(f, qb)
            pltpu.sync_copy(msg.at[qb], acc.at[fc.at[qb]], add=True)

        @pl.when(f >= 1)
        def _():
            pb = (f - 1) % 3
            _gdrain(f - 1, pb)
            pltpu.sync_copy(msg.at[pb], acc.at[fc.at[pb]], add=True)

        plsc.subcore_barrier()
        pltpu.sync_copy(acc.at[pl.ds(zbase, ZR)],
                        out_hbm.at[cid, pl.ds(zbase, ZR)])

    return prop


def _make_deg():
    mesh = plsc.VectorSubcoreMesh(core_axis_name="c", subcore_axis_name="s")

    @functools.partial(
        pl.kernel,
        mesh=mesh,
        out_type=jax.ShapeDtypeStruct((NC, DEG_R, DEG_W), jnp.float32),
        scratch_types=[
            pltpu.VMEM((NCH_D, CH), jnp.int32),
            pltpu.VMEM((CH, DEG_W), jnp.float32),
            pltpu.VMEM_SHARED((DEG_R, DEG_W), jnp.float32),
            pltpu.SemaphoreType.DMA,
        ],
        compiler_params=pltpu.CompilerParams(use_tc_tiling_on_sc=False),
    )
    def deg(rc_hbm, ones_hbm, zeros_hbm, out_hbm, idx, ones_v, acc, dsem):
        cid = lax.axis_index("c")
        sid = lax.axis_index("s")
        pltpu.sync_copy(rc_hbm.at[cid, sid], idx)
        pltpu.sync_copy(ones_hbm, ones_v)
        dbase = sid * DZR
        pltpu.sync_copy(zeros_hbm, acc.at[pl.ds(dbase, DZR)])
        plsc.subcore_barrier()

        def dloop(j, _):
            @pl.when(j >= 16)
            def _():
                pltpu.make_async_copy(ones_hbm, ones_v, dsem).wait()

            pltpu.async_copy(ones_v, acc.at[idx.at[j]], dsem, add=True)
            return 0

        lax.fori_loop(0, NCH_D, dloop, 0)

        def ddrain(j, _):
            pltpu.make_async_copy(ones_hbm, ones_v, dsem).wait()
            return 0

        lax.fori_loop(0, 16, ddrain, 0)
        plsc.subcore_barrier()
        pltpu.sync_copy(acc.at[pl.ds(dbase, DZR)],
                        out_hbm.at[cid, pl.ds(dbase, DZR)])

    return deg


_prop = _make_prop()
_deg = _make_deg()


def _init_body(x_ref, w_ref, b_ref, d_ref, h_ref, hs_ref, dinv_ref):
    degree = d_ref[0, :, 0:1] + d_ref[1, :, 0:1] + 1.0
    dinv = lax.rsqrt(degree)
    h = jnp.dot(x_ref[...], w_ref[...],
                preferred_element_type=jnp.float32) + b_ref[...]
    h = jnp.where(h >= 0, h, 0.01 * h)
    h_ref[...] = h
    hs_ref[...] = dinv * h
    dinv_ref[...] = dinv


def _tc_init(xp, w0p, b0r, dparts):
    return pl.pallas_call(
        _init_body,
        grid=(NBLK,),
        in_specs=[
            pl.BlockSpec((BLK, 8), lambda b: (b, 0)),
            pl.BlockSpec((8, H), lambda b: (0, 0)),
            pl.BlockSpec((1, H), lambda b: (0, 0)),
            pl.BlockSpec((NC, BLK, DEG_W), lambda b: (0, b, 0)),
        ],
        out_specs=[
            pl.BlockSpec((BLK, H), lambda b: (b, 0)),
            pl.BlockSpec((BLK, H), lambda b: (b, 0)),
            pl.BlockSpec((BLK, 1), lambda b: (b, 0)),
        ],
        out_shape=[
            jax.ShapeDtypeStruct((N, H), jnp.float32),
            jax.ShapeDtypeStruct((N, H), jnp.float32),
            jax.ShapeDtypeStruct((N, 1), jnp.float32),
        ],
    )(xp, w0p, b0r, dparts)


def _gru_body(a_ref, h_ref, dinv_ref, m_ref, wir, wiz, win, whr, whz, whn,
              br, bz, bni, bnh, hn_ref, hs_ref, mo_ref):
    dinv = dinv_ref[...]
    x = dinv * a_ref[0]
    h = h_ref[...]

    def mm(v, w):
        return jnp.dot(v, w[...], preferred_element_type=jnp.float32)

    r = jax.nn.sigmoid(mm(x, wir) + mm(h, whr) + br[...])
    z = jax.nn.sigmoid(mm(x, wiz) + mm(h, whz) + bz[...])
    n = jnp.tanh(mm(x, win) + bni[...] + r * (mm(h, whn) + bnh[...]))
    hn = (1.0 - z) * n + z * h
    hn_ref[...] = hn
    hs_ref[...] = dinv * hn
    mo_ref[...] = jnp.maximum(m_ref[...], hn)


def _tc_gru(agg, h, dinv, m, ws, bs):
    wfull = pl.BlockSpec((H, H), lambda b: (0, 0))
    bfull = pl.BlockSpec((1, H), lambda b: (0, 0))
    nodes = pl.BlockSpec((BLK, H), lambda b: (b, 0))
    return pl.pallas_call(
        _gru_body,
        grid=(NBLK,),
        in_specs=[
            pl.BlockSpec((1, BLK, H), lambda b: (b // (HALF // BLK),
                                                 b % (HALF // BLK), 0)),
            nodes,
            pl.BlockSpec((BLK, 1), lambda b: (b, 0)),
            nodes,
        ] + [wfull] * 6 + [bfull] * 4,
        out_specs=[nodes, nodes, nodes],
        out_shape=[
            jax.ShapeDtypeStruct((N, H), jnp.float32),
            jax.ShapeDtypeStruct((N, H), jnp.float32),
            jax.ShapeDtypeStruct((N, H), jnp.float32),
        ],
    )(agg, h, dinv, m, *ws, *bs)


def _gru_fin_body(a_ref, h_ref, dinv_ref, m_ref, wir, wiz, win, whr, whz,
                  whn, br, bz, bni, bnh, w1_ref, b1_ref, w2_ref, b2_ref,
                  o_ref):
    dinv = dinv_ref[...]
    x = dinv * a_ref[0]
    h = h_ref[...]

    def mm(v, w):
        return jnp.dot(v, w[...], preferred_element_type=jnp.float32)

    r = jax.nn.sigmoid(mm(x, wir) + mm(h, whr) + br[...])
    z = jax.nn.sigmoid(mm(x, wiz) + mm(h, whz) + bz[...])
    n = jnp.tanh(mm(x, win) + bni[...] + r * (mm(h, whn) + bnh[...]))
    hn = (1.0 - z) * n + z * h
    mx = jnp.maximum(m_ref[...], hn)
    t = jnp.dot(mx, w1_ref[...], preferred_element_type=jnp.float32) + b1_ref[...]
    t = jnp.where(t >= 0, t, 0.01 * t)
    o_ref[...] = jnp.sum(t * w2_ref[...], axis=1, keepdims=True) + b2_ref[...]


def _tc_gru_fin(agg, h, dinv, m, ws, bs, w1, b1r, w2r, b2r):
    wfull = pl.BlockSpec((H, H), lambda b: (0, 0))
    bfull = pl.BlockSpec((1, H), lambda b: (0, 0))
    nodes = pl.BlockSpec((BLK, H), lambda b: (b, 0))
    return pl.pallas_call(
        _gru_fin_body,
        grid=(NBLK,),
        in_specs=[
            pl.BlockSpec((1, BLK, H), lambda b: (b // (HALF // BLK),
                                                 b % (HALF // BLK), 0)),
            nodes,
            pl.BlockSpec((BLK, 1), lambda b: (b, 0)),
            nodes,
        ] + [wfull] * 6 + [bfull] * 4 + [
            pl.BlockSpec((H, H // 2), lambda b: (0, 0)),
            pl.BlockSpec((1, H // 2), lambda b: (0, 0)),
            pl.BlockSpec((1, H // 2), lambda b: (0, 0)),
            pl.BlockSpec((1, 1), lambda b: (0, 0)),
        ],
        out_specs=pl.BlockSpec((BLK, 1), lambda b: (b, 0)),
        out_shape=jax.ShapeDtypeStruct((N, 1), jnp.float32),
    )(agg, h, dinv, m, *ws, *bs, w1, b1r, w2r, b2r)


def _fin_body(m_ref, w1_ref, b1_ref, w2_ref, b2_ref, o_ref):
    t = jnp.dot(m_ref[...], w1_ref[...],
                preferred_element_type=jnp.float32) + b1_ref[...]
    t = jnp.where(t >= 0, t, 0.01 * t)
    o_ref[...] = jnp.sum(t * w2_ref[...], axis=1, keepdims=True) + b2_ref[...]


def _tc_final(m, w1, b1r, w2r, b2r):
    return pl.pallas_call(
        _fin_body,
        grid=(NBLK,),
        in_specs=[
            pl.BlockSpec((BLK, H), lambda b: (b, 0)),
            pl.BlockSpec((H, H // 2), lambda b: (0, 0)),
            pl.BlockSpec((1, H // 2), lambda b: (0, 0)),
            pl.BlockSpec((1, H // 2), lambda b: (0, 0)),
            pl.BlockSpec((1, 1), lambda b: (0, 0)),
        ],
        out_specs=pl.BlockSpec((BLK, 1), lambda b: (b, 0)),
        out_shape=jax.ShapeDtypeStruct((N, 1), jnp.float32),
    )(m, w1, b1r, w2r, b2r)


def kernel(X, edge_index, W0, b0, Wih, Whh, bih, bhh, W1, b1, W2, b2):
    row = edge_index[0]
    col = edge_index[1]

    rowp = jnp.concatenate(
        [row, jnp.zeros((EPAD - E,), jnp.int32)]).reshape(NS, NCH, CH)
    colp = jnp.concatenate(
        [col, jnp.full((EPAD - E,), 1 << 29, jnp.int32)]).reshape(NS, NCH, CH)
    rc = jnp.concatenate([row, col,
                          jnp.full((DPAD - 2 * E,), TRASH_D, jnp.int32)])
    rc_t = rc.reshape(NC, NS, NCH_D, CH)

    zb = jnp.zeros((ZCH, H), jnp.float32)
    ones1 = jnp.ones((CH, DEG_W), jnp.float32)
    zeros_d = jnp.zeros((DZR, DEG_W), jnp.float32)

    xp = jnp.pad(X, ((0, 0), (0, 8 - X.shape[1])))
    w0p = jnp.pad(W0, ((0, 8 - W0.shape[0]), (0, 0)))
    b0r = b0.reshape(1, H)
    wt_i = Wih.T
    wt_h = Whh.T
    ws = (wt_i[:, 0:H], wt_i[:, H:2 * H], wt_i[:, 2 * H:3 * H],
          wt_h[:, 0:H], wt_h[:, H:2 * H], wt_h[:, 2 * H:3 * H])
    bs = ((bih[0:H] + bhh[0:H]).reshape(1, H),
          (bih[H:2 * H] + bhh[H:2 * H]).reshape(1, H),
          bih[2 * H:3 * H].reshape(1, H),
          bhh[2 * H:3 * H].reshape(1, H))
    b1r = b1.reshape(1, H // 2)
    w2r = W2.T
    b2r = b2.reshape(1, 1)

    dparts = _deg(rc_t, ones1, zeros_d)

    h, hs, dinv = _tc_init(xp, w0p, b0r, dparts)

    m = jnp.full((N, H), -jnp.inf, jnp.float32)
    for _ in range(DEPTH - 2):
        agg = _prop(hs, rowp, colp, zb)
        h, hs, m = _tc_gru(agg, h, dinv, m, ws, bs)
    agg = _prop(hs, rowp, colp, zb)
    out = _tc_gru_fin(agg, h, dinv, m, ws, bs, W1, b1r, w2r, b2r)
    return out.reshape(N)


# speedup vs baseline: 1.0127x; 1.0127x over previous
import functools

import jax
import jax.numpy as jnp
from jax import lax
from jax.experimental import pallas as pl
from jax.experimental.pallas import tpu as pltpu
from jax.experimental.pallas import tpu_sc as plsc

N = 50000
E = 800000
H = 64
DEPTH = 6

NC = 2
NS = 16
CH = 128
HALF = N // NC

NCH = 392
G = 14
NG = NCH // G
EP_TILE = NCH * CH
EPAD = NS * EP_TILE

RACC = 25088
ZR = RACC // NS
ZCH = 112
TRASH_P = 25024

NCH_D = 392
DPAD = NC * NS * NCH_D * CH
DEG_R = 51200
DZR = DEG_R // NS
TRASH_D = 50176
DEG_W = 16

BLK = 1000
NBLK = N // BLK


def _make_prop():
    mesh = plsc.VectorSubcoreMesh(core_axis_name="c", subcore_axis_name="s")

    @functools.partial(
        pl.kernel,
        mesh=mesh,
        out_type=jax.ShapeDtypeStruct((NC, RACC, H), jnp.float32),
        scratch_types=[
            pltpu.VMEM((G, CH), jnp.int32),
            pltpu.VMEM((G, CH), jnp.int32),
            pltpu.VMEM((256,), jnp.int32),
            pltpu.VMEM((256,), jnp.int32),
            pltpu.VMEM((3, CH), jnp.int32),
            pltpu.VMEM((3, CH), jnp.int32),
            pltpu.VMEM((3, CH, H), jnp.float32),
            pltpu.VMEM_SHARED((RACC, H), jnp.float32),
            pltpu.SemaphoreType.DMA,
            pltpu.SemaphoreType.DMA,
            pltpu.SemaphoreType.DMA,
        ],
        compiler_params=pltpu.CompilerParams(use_tc_tiling_on_sc=False,
                                             needs_layout_passes=False),
    )
    def prop(hs_hbm, row_hbm, col_hbm, zb_hbm, out_hbm, rowi, coli, wr, wc,
             fr, fc, msg, acc, gs0, gs1, ssem):
        cid = lax.axis_index("c")
        sid = lax.axis_index("s")
        lo = cid * HALF
        zbase = sid * ZR

        def zloop(j, _):
            pltpu.async_copy(zb_hbm, acc.at[pl.ds(zbase + j * ZCH, ZCH)],
                             ssem)
            return 0

        lax.fori_loop(0, ZR // ZCH, zloop, 0)

        def zdrain(j, _):
            pltpu.make_async_copy(zb_hbm, acc.at[pl.ds(zbase + j * ZCH, ZCH)],
                                  ssem).wait()
            return 0

        lax.fori_loop(0, ZR // ZCH, zdrain, 0)
        plsc.subcore_barrier()

        def _drain_msg(b, sem):
            pltpu.make_async_copy(zb_hbm, msg.at[b, pl.ds(0, ZCH)],
                                  sem).wait()
            pltpu.make_async_copy(zb_hbm, msg.at[b, pl.ds(ZCH, CH - ZCH)],
                                  sem).wait()

        def _gdrain(fpar, b):
            def even():
                _drain_msg(b, gs0)
                return 0

            def odd():
                _drain_msg(b, gs1)
                return 0

            lax.cond(fpar % 2 == 0, even, odd)

        def _gissue(fpar, fb):
            def even():
                pltpu.async_copy(hs_hbm.at[fr.at[fb]], msg.at[fb], gs0)
                return 0

            def odd():
                pltpu.async_copy(hs_hbm.at[fr.at[fb]], msg.at[fb], gs1)
                return 0

            lax.cond(fpar % 2 == 0, even, odd)

        def fire(off, f):
            fb = f % 3
            for m in range(8):
                sl = pl.ds(16 * m, 16)
                fr[fb, sl] = wr[sl]
                fc[fb, sl] = wc[sl]
            for m in range(8):
                fr_sl = pl.ds(128 + 16 * m, 16)
                to_sl = pl.ds(16 * m, 16)
                wr[to_sl] = wr[fr_sl]
                wc[to_sl] = wc[fr_sl]

            @pl.when(f >= 3)
            def _():
                _drain_msg((f - 3) % 3, ssem)

            @pl.when(f >= 2)
            def _():
                qb = (f - 2) % 3
                _gdrain(f, qb)
                pltpu.async_copy(msg.at[qb], acc.at[fc.at[qb]], ssem,
                                 add=True)

            _gissue(f, fb)
            return off - 128, f + 1

        def gloop(g, carry):
            off, f = carry
            pltpu.sync_copy(row_hbm.at[sid, pl.ds(g * G, G)], rowi)
            pltpu.sync_copy(col_hbm.at[sid, pl.ds(g * G, G)], coli)
            for j in range(G):
                for k in range(8):
                    sl = pl.ds(16 * k, 16)
                    rv = rowi[j, sl]
                    cv = coli[j, sl]
                    msk = (cv >= lo) & (cv < lo + HALF)
                    loc = cv - lo
                    plsc.store_compressed(wr.at[pl.ds(off, 16)], rv, mask=msk)
                    plsc.store_compressed(wc.at[pl.ds(off, 16)], loc, mask=msk)
                    off = off + jnp.sum(jnp.where(msk, 1, 0))
                off, f = lax.cond(off >= 128, fire, lambda o, ff: (o, ff),
                                  off, f)
            return off, f

        off, f = lax.fori_loop(0, NG, gloop, (jnp.int32(0), jnp.int32(0)))

        iota16 = lax.iota(jnp.int32, 16)
        for m in range(8):
            sl = pl.ds(16 * m, 16)
            pos = 16 * m + iota16
            tail = pos >= off
            wr[sl] = jnp.where(tail, 0, wr[sl])
            wc[sl] = jnp.where(tail, TRASH_P + (pos & 63), wc[sl])
        off, f = fire(off, f)

        @pl.when(f >= 3)
        def _():
            _drain_msg((f - 3) % 3, ssem)

        @pl.when(f >= 2)
        def _():
            qb = (f - 2) % 3
            _gdrain(f, qb)
            pltpu.sync_copy(msg.at[qb], acc.at[fc.at[qb]], add=True)

        @pl.when(f >= 1)
        def _():
            pb = (f - 1) % 3
            _gdrain(f - 1, pb)
            pltpu.sync_copy(msg.at[pb], acc.at[fc.at[pb]], add=True)

        plsc.subcore_barrier()
        pltpu.sync_copy(acc.at[pl.ds(zbase, ZR)],
                        out_hbm.at[cid, pl.ds(zbase, ZR)])

    return prop


def _make_deg():
    mesh = plsc.VectorSubcoreMesh(core_axis_name="c", subcore_axis_name="s")

    @functools.partial(
        pl.kernel,
        mesh=mesh,
        out_type=jax.ShapeDtypeStruct((NC, DEG_R, DEG_W), jnp.float32),
        scratch_types=[
            pltpu.VMEM((NCH_D, CH), jnp.int32),
            pltpu.VMEM((CH, DEG_W), jnp.float32),
            pltpu.VMEM_SHARED((DEG_R, DEG_W), jnp.float32),
            pltpu.SemaphoreType.DMA,
        ],
        compiler_params=pltpu.CompilerParams(use_tc_tiling_on_sc=False),
    )
    def deg(rc_hbm, ones_hbm, zeros_hbm, out_hbm, idx, ones_v, acc, dsem):
        cid = lax.axis_index("c")
        sid = lax.axis_index("s")
        pltpu.sync_copy(rc_hbm.at[cid, sid], idx)
        pltpu.sync_copy(ones_hbm, ones_v)
        dbase = sid * DZR
        pltpu.sync_copy(zeros_hbm, acc.at[pl.ds(dbase, DZR)])
        plsc.subcore_barrier()

        def dloop(j, _):
            @pl.when(j >= 16)
            def _():
                pltpu.make_async_copy(ones_hbm, ones_v, dsem).wait()

            pltpu.async_copy(ones_v, acc.at[idx.at[j]], dsem, add=True)
            return 0

        lax.fori_loop(0, NCH_D, dloop, 0)

        def ddrain(j, _):
            pltpu.make_async_copy(ones_hbm, ones_v, dsem).wait()
            return 0

        lax.fori_loop(0, 16, ddrain, 0)
        plsc.subcore_barrier()
        pltpu.sync_copy(acc.at[pl.ds(dbase, DZR)],
                        out_hbm.at[cid, pl.ds(dbase, DZR)])

    return deg


_prop = _make_prop()
_deg = _make_deg()


def _init_body(x_ref, w_ref, b_ref, d_ref, h_ref, hs_ref, dinv_ref):
    degree = d_ref[0, :, 0:1] + d_ref[1, :, 0:1] + 1.0
    dinv = lax.rsqrt(degree)
    h = jnp.dot(x_ref[...], w_ref[...],
                preferred_element_type=jnp.float32) + b_ref[...]
    h = jnp.where(h >= 0, h, 0.01 * h)
    h_ref[...] = h
    hs_ref[...] = dinv * h
    dinv_ref[...] = dinv


def _tc_init(xp, w0p, b0r, dparts):
    return pl.pallas_call(
        _init_body,
        grid=(NBLK,),
        in_specs=[
            pl.BlockSpec((BLK, 8), lambda b: (b, 0)),
            pl.BlockSpec((8, H), lambda b: (0, 0)),
            pl.BlockSpec((1, H), lambda b: (0, 0)),
            pl.BlockSpec((NC, BLK, DEG_W), lambda b: (0, b, 0)),
        ],
        out_specs=[
            pl.BlockSpec((BLK, H), lambda b: (b, 0)),
            pl.BlockSpec((BLK, H), lambda b: (b, 0)),
            pl.BlockSpec((BLK, 1), lambda b: (b, 0)),
        ],
        out_shape=[
            jax.ShapeDtypeStruct((N, H), jnp.float32),
            jax.ShapeDtypeStruct((N, H), jnp.float32),
            jax.ShapeDtypeStruct((N, 1), jnp.float32),
        ],
    )(xp, w0p, b0r, dparts)


def _gru_body(a_ref, h_ref, dinv_ref, m_ref, wir, wiz, win, whr, whz, whn,
              br, bz, bni, bnh, hn_ref, hs_ref, mo_ref):
    dinv = dinv_ref[...]
    x = dinv * a_ref[0]
    h = h_ref[...]

    def mm(v, w):
        return jnp.dot(v, w[...], preferred_element_type=jnp.float32)

    r = jax.nn.sigmoid(mm(x, wir) + mm(h, whr) + br[...])
    z = jax.nn.sigmoid(mm(x, wiz) + mm(h, whz) + bz[...])
    n = jnp.tanh(mm(x, win) + bni[...] + r * (mm(h, whn) + bnh[...]))
    hn = (1.0 - z) * n + z * h
    hn_ref[...] = hn
    hs_ref[...] = dinv * hn
    mo_ref[...] = jnp.maximum(m_ref[...], hn)


def _tc_gru(agg, h, dinv, m, ws, bs):
    wfull = pl.BlockSpec((H, H), lambda b: (0, 0))
    bfull = pl.BlockSpec((1, H), lambda b: (0, 0))
    nodes = pl.BlockSpec((BLK, H), lambda b: (b, 0))
    return pl.pallas_call(
        _gru_body,
        grid=(NBLK,),
        in_specs=[
            pl.BlockSpec((1, BLK, H), lambda b: (b // (HALF // BLK),
                                                 b % (HALF // BLK), 0)),
            nodes,
            pl.BlockSpec((BLK, 1), lambda b: (b, 0)),
            nodes,
        ] + [wfull] * 6 + [bfull] * 4,
        out_specs=[nodes, nodes, nodes],
        out_shape=[
            jax.ShapeDtypeStruct((N, H), jnp.float32),
            jax.ShapeDtypeStruct((N, H), jnp.float32),
            jax.ShapeDtypeStruct((N, H), jnp.float32),
        ],
    )(agg, h, dinv, m, *ws, *bs)


def _fin_body(m_ref, w1_ref, b1_ref, w2_ref, b2_ref, o_ref):
    t = jnp.dot(m_ref[...], w1_ref[...],
                preferred_element_type=jnp.float32) + b1_ref[...]
    t = jnp.where(t >= 0, t, 0.01 * t)
    o_ref[...] = jnp.sum(t * w2_ref[...], axis=1, keepdims=True) + b2_ref[...]


def _tc_final(m, w1, b1r, w2r, b2r):
    return pl.pallas_call(
        _fin_body,
        grid=(NBLK,),
        in_specs=[
            pl.BlockSpec((BLK, H), lambda b: (b, 0)),
            pl.BlockSpec((H, H // 2), lambda b: (0, 0)),
            pl.BlockSpec((1, H // 2), lambda b: (0, 0)),
            pl.BlockSpec((1, H // 2), lambda b: (0, 0)),
            pl.BlockSpec((1, 1), lambda b: (0, 0)),
        ],
        out_specs=pl.BlockSpec((BLK, 1), lambda b: (b, 0)),
        out_shape=jax.ShapeDtypeStruct((N, 1), jnp.float32),
    )(m, w1, b1r, w2r, b2r)


def kernel(X, edge_index, W0, b0, Wih, Whh, bih, bhh, W1, b1, W2, b2):
    row = edge_index[0]
    col = edge_index[1]

    rowp = jnp.concatenate(
        [row, jnp.zeros((EPAD - E,), jnp.int32)]).reshape(NS, NCH, CH)
    colp = jnp.concatenate(
        [col, jnp.full((EPAD - E,), 1 << 29, jnp.int32)]).reshape(NS, NCH, CH)
    rc = jnp.concatenate([row, col,
                          jnp.full((DPAD - 2 * E,), TRASH_D, jnp.int32)])
    rc_t = rc.reshape(NC, NS, NCH_D, CH)

    zb = jnp.zeros((ZCH, H), jnp.float32)
    ones1 = jnp.ones((CH, DEG_W), jnp.float32)
    zeros_d = jnp.zeros((DZR, DEG_W), jnp.float32)

    xp = jnp.pad(X, ((0, 0), (0, 8 - X.shape[1])))
    w0p = jnp.pad(W0, ((0, 8 - W0.shape[0]), (0, 0)))
    b0r = b0.reshape(1, H)
    wt_i = Wih.T
    wt_h = Whh.T
    ws = (wt_i[:, 0:H], wt_i[:, H:2 * H], wt_i[:, 2 * H:3 * H],
          wt_h[:, 0:H], wt_h[:, H:2 * H], wt_h[:, 2 * H:3 * H])
    bs = ((bih[0:H] + bhh[0:H]).reshape(1, H),
          (bih[H:2 * H] + bhh[H:2 * H]).reshape(1, H),
          bih[2 * H:3 * H].reshape(1, H),
          bhh[2 * H:3 * H].reshape(1, H))
    b1r = b1.reshape(1, H // 2)
    w2r = W2.T
    b2r = b2.reshape(1, 1)

    dparts = _deg(rc_t, ones1, zeros_d)

    h, hs, dinv = _tc_init(xp, w0p, b0r, dparts)

    m = jnp.full((N, H), -jnp.inf, jnp.float32)
    for _ in range(DEPTH - 1):
        agg = _prop(hs, rowp, colp, zb)
        h, hs, m = _tc_gru(agg, h, dinv, m, ws, bs)

    out = _tc_final(m, W1, b1r, w2r, b2r)
    return out.reshape(N)


# speedup vs baseline: 1.0934x; 1.0797x over previous
import functools

import jax
import jax.numpy as jnp
from jax import lax
from jax.experimental import pallas as pl
from jax.experimental.pallas import tpu as pltpu
from jax.experimental.pallas import tpu_sc as plsc

N = 50000
E = 800000
H = 64
DEPTH = 6

NC = 2
NS = 16
CH = 128
HALF = N // NC

NCH = 392
G = 14
NG = NCH // G
EP_TILE = NCH * CH
EPAD = NS * EP_TILE

RACC = 25088
ZR = RACC // NS
ZCH = 112
TRASH_P = 25024

NCH_D = 392
DPAD = NC * NS * NCH_D * CH
DEG_R = 51200
DZR = DEG_R // NS
TRASH_D = 50176
DEG_W = 16

BLK = 5000
NBLK = N // BLK


def _make_prop():
    mesh = plsc.VectorSubcoreMesh(core_axis_name="c", subcore_axis_name="s")

    @functools.partial(
        pl.kernel,
        mesh=mesh,
        out_type=jax.ShapeDtypeStruct((NC, RACC, H), jnp.float32),
        scratch_types=[
            pltpu.VMEM((G, CH), jnp.int32),
            pltpu.VMEM((G, CH), jnp.int32),
            pltpu.VMEM((256,), jnp.int32),
            pltpu.VMEM((256,), jnp.int32),
            pltpu.VMEM((3, CH), jnp.int32),
            pltpu.VMEM((3, CH), jnp.int32),
            pltpu.VMEM((3, CH, H), jnp.float32),
            pltpu.VMEM_SHARED((RACC, H), jnp.float32),
            pltpu.SemaphoreType.DMA,
            pltpu.SemaphoreType.DMA,
            pltpu.SemaphoreType.DMA,
        ],
        compiler_params=pltpu.CompilerParams(use_tc_tiling_on_sc=False,
                                             needs_layout_passes=False),
    )
    def prop(hs_hbm, row_hbm, col_hbm, zb_hbm, out_hbm, rowi, coli, wr, wc,
             fr, fc, msg, acc, gs0, gs1, ssem):
        cid = lax.axis_index("c")
        sid = lax.axis_index("s")
        lo = cid * HALF
        zbase = sid * ZR

        def zloop(j, _):
            pltpu.async_copy(zb_hbm, acc.at[pl.ds(zbase + j * ZCH, ZCH)],
                             ssem)
            return 0

        lax.fori_loop(0, ZR // ZCH, zloop, 0)

        def zdrain(j, _):
            pltpu.make_async_copy(zb_hbm, acc.at[pl.ds(zbase + j * ZCH, ZCH)],
                                  ssem).wait()
            return 0

        lax.fori_loop(0, ZR // ZCH, zdrain, 0)
        plsc.subcore_barrier()

        def _drain_msg(b, sem):
            pltpu.make_async_copy(zb_hbm, msg.at[b, pl.ds(0, ZCH)],
                                  sem).wait()
            pltpu.make_async_copy(zb_hbm, msg.at[b, pl.ds(ZCH, CH - ZCH)],
                                  sem).wait()

        def _gdrain(fpar, b):
            def even():
                _drain_msg(b, gs0)
                return 0

            def odd():
                _drain_msg(b, gs1)
                return 0

            lax.cond(fpar % 2 == 0, even, odd)

        def _gissue(fpar, fb):
            def even():
                pltpu.async_copy(hs_hbm.at[fr.at[fb]], msg.at[fb], gs0)
                return 0

            def odd():
                pltpu.async_copy(hs_hbm.at[fr.at[fb]], msg.at[fb], gs1)
                return 0

            lax.cond(fpar % 2 == 0, even, odd)

        def fire(off, f):
            fb = f % 3
            for m in range(8):
                sl = pl.ds(16 * m, 16)
                fr[fb, sl] = wr[sl]
                fc[fb, sl] = wc[sl]
            for m in range(8):
                fr_sl = pl.ds(128 + 16 * m, 16)
                to_sl = pl.ds(16 * m, 16)
                wr[to_sl] = wr[fr_sl]
                wc[to_sl] = wc[fr_sl]

            @pl.when(f >= 3)
            def _():
                _drain_msg((f - 3) % 3, ssem)

            @pl.when(f >= 2)
            def _():
                qb = (f - 2) % 3
                _gdrain(f, qb)
                pltpu.async_copy(msg.at[qb], acc.at[fc.at[qb]], ssem,
                                 add=True)

            _gissue(f, fb)
            return off - 128, f + 1

        def gloop(g, carry):
            off, f = carry
            pltpu.sync_copy(row_hbm.at[sid, pl.ds(g * G, G)], rowi)
            pltpu.sync_copy(col_hbm.at[sid, pl.ds(g * G, G)], coli)
            for j in range(G):
                for k in range(8):
                    sl = pl.ds(16 * k, 16)
                    rv = rowi[j, sl]
                    cv = coli[j, sl]
                    msk = (cv >= lo) & (cv < lo + HALF)
                    loc = cv - lo
                    plsc.store_compressed(wr.at[pl.ds(off, 16)], rv, mask=msk)
                    plsc.store_compressed(wc.at[pl.ds(off, 16)], loc, mask=msk)
                    off = off + jnp.sum(jnp.where(msk, 1, 0))
                off, f = lax.cond(off >= 128, fire, lambda o, ff: (o, ff),
                                  off, f)
            return off, f

        off, f = lax.fori_loop(0, NG, gloop, (jnp.int32(0), jnp.int32(0)))

        iota16 = lax.iota(jnp.int32, 16)
        for m in range(8):
            sl = pl.ds(16 * m, 16)
            pos = 16 * m + iota16
            tail = pos >= off
            wr[sl] = jnp.where(tail, 0, wr[sl])
            wc[sl] = jnp.where(tail, TRASH_P + (pos & 63), wc[sl])
        off, f = fire(off, f)

        @pl.when(f >= 3)
        def _():
            _drain_msg((f - 3) % 3, ssem)

        @pl.when(f >= 2)
        def _():
            qb = (f - 2) % 3
            _gdrain(f, qb)
            pltpu.sync_copy(msg.at[qb], acc.at[fc.at[qb]], add=True)

        @pl.when(f >= 1)
        def _():
            pb = (f - 1) % 3
            _gdrain(f - 1, pb)
            pltpu.sync_copy(msg.at[pb], acc.at[fc.at[pb]], add=True)

        plsc.subcore_barrier()
        pltpu.sync_copy(acc.at[pl.ds(zbase, ZR)],
                        out_hbm.at[cid, pl.ds(zbase, ZR)])

    return prop


def _make_deg():
    mesh = plsc.VectorSubcoreMesh(core_axis_name="c", subcore_axis_name="s")

    @functools.partial(
        pl.kernel,
        mesh=mesh,
        out_type=jax.ShapeDtypeStruct((NC, DEG_R, DEG_W), jnp.float32),
        scratch_types=[
            pltpu.VMEM((NCH_D, CH), jnp.int32),
            pltpu.VMEM((CH, DEG_W), jnp.float32),
            pltpu.VMEM_SHARED((DEG_R, DEG_W), jnp.float32),
            pltpu.SemaphoreType.DMA,
        ],
        compiler_params=pltpu.CompilerParams(use_tc_tiling_on_sc=False),
    )
    def deg(rc_hbm, ones_hbm, zeros_hbm, out_hbm, idx, ones_v, acc, dsem):
        cid = lax.axis_index("c")
        sid = lax.axis_index("s")
        pltpu.sync_copy(rc_hbm.at[cid, sid], idx)
        pltpu.sync_copy(ones_hbm, ones_v)
        dbase = sid * DZR
        pltpu.sync_copy(zeros_hbm, acc.at[pl.ds(dbase, DZR)])
        plsc.subcore_barrier()

        def dloop(j, _):
            @pl.when(j >= 16)
            def _():
                pltpu.make_async_copy(ones_hbm, ones_v, dsem).wait()

            pltpu.async_copy(ones_v, acc.at[idx.at[j]], dsem, add=True)
            return 0

        lax.fori_loop(0, NCH_D, dloop, 0)

        def ddrain(j, _):
            pltpu.make_async_copy(ones_hbm, ones_v, dsem).wait()
            return 0

        lax.fori_loop(0, 16, ddrain, 0)
        plsc.subcore_barrier()
        pltpu.sync_copy(acc.at[pl.ds(dbase, DZR)],
                        out_hbm.at[cid, pl.ds(dbase, DZR)])

    return deg


_prop = _make_prop()
_deg = _make_deg()


def _init_body(x_ref, w_ref, b_ref, d_ref, h_ref, hs_ref, dinv_ref):
    degree = d_ref[0, :, 0:1] + d_ref[1, :, 0:1] + 1.0
    dinv = lax.rsqrt(degree)
    h = jnp.dot(x_ref[...], w_ref[...],
                preferred_element_type=jnp.float32) + b_ref[...]
    h = jnp.where(h >= 0, h, 0.01 * h)
    h_ref[...] = h
    hs_ref[...] = dinv * h
    dinv_ref[...] = dinv


def _tc_init(xp, w0p, b0r, dparts):
    return pl.pallas_call(
        _init_body,
        grid=(NBLK,),
        in_specs=[
            pl.BlockSpec((BLK, 8), lambda b: (b, 0)),
            pl.BlockSpec((8, H), lambda b: (0, 0)),
            pl.BlockSpec((1, H), lambda b: (0, 0)),
            pl.BlockSpec((NC, BLK, DEG_W), lambda b: (0, b, 0)),
        ],
        out_specs=[
            pl.BlockSpec((BLK, H), lambda b: (b, 0)),
            pl.BlockSpec((BLK, H), lambda b: (b, 0)),
            pl.BlockSpec((BLK, 1), lambda b: (b, 0)),
        ],
        out_shape=[
            jax.ShapeDtypeStruct((N, H), jnp.float32),
            jax.ShapeDtypeStruct((N, H), jnp.float32),
            jax.ShapeDtypeStruct((N, 1), jnp.float32),
        ],
    )(xp, w0p, b0r, dparts)


def _gru_body(a_ref, h_ref, dinv_ref, m_ref, wir, wiz, win, whr, whz, whn,
              br, bz, bni, bnh, hn_ref, hs_ref, mo_ref):
    dinv = dinv_ref[...]
    x = dinv * a_ref[0]
    h = h_ref[...]

    def mm(v, w):
        return jnp.dot(v, w[...], preferred_element_type=jnp.float32)

    r = jax.nn.sigmoid(mm(x, wir) + mm(h, whr) + br[...])
    z = jax.nn.sigmoid(mm(x, wiz) + mm(h, whz) + bz[...])
    n = jnp.tanh(mm(x, win) + bni[...] + r * (mm(h, whn) + bnh[...]))
    hn = (1.0 - z) * n + z * h
    hn_ref[...] = hn
    hs_ref[...] = dinv * hn
    mo_ref[...] = jnp.maximum(m_ref[...], hn)


def _tc_gru(agg, h, dinv, m, ws, bs):
    wfull = pl.BlockSpec((H, H), lambda b: (0, 0))
    bfull = pl.BlockSpec((1, H), lambda b: (0, 0))
    nodes = pl.BlockSpec((BLK, H), lambda b: (b, 0))
    return pl.pallas_call(
        _gru_body,
        grid=(NBLK,),
        in_specs=[
            pl.BlockSpec((1, BLK, H), lambda b: (b // (HALF // BLK),
                                                 b % (HALF // BLK), 0)),
            nodes,
            pl.BlockSpec((BLK, 1), lambda b: (b, 0)),
            nodes,
        ] + [wfull] * 6 + [bfull] * 4,
        out_specs=[nodes, nodes, nodes],
        out_shape=[
            jax.ShapeDtypeStruct((N, H), jnp.float32),
            jax.ShapeDtypeStruct((N, H), jnp.float32),
            jax.ShapeDtypeStruct((N, H), jnp.float32),
        ],
    )(agg, h, dinv, m, *ws, *bs)


def _fin_body(m_ref, w1_ref, b1_ref, w2_ref, b2_ref, o_ref):
    t = jnp.dot(m_ref[...], w1_ref[...],
                preferred_element_type=jnp.float32) + b1_ref[...]
    t = jnp.where(t >= 0, t, 0.01 * t)
    o_ref[...] = jnp.sum(t * w2_ref[...], axis=1, keepdims=True) + b2_ref[...]


def _tc_final(m, w1, b1r, w2r, b2r):
    return pl.pallas_call(
        _fin_body,
        grid=(NBLK,),
        in_specs=[
            pl.BlockSpec((BLK, H), lambda b: (b, 0)),
            pl.BlockSpec((H, H // 2), lambda b: (0, 0)),
            pl.BlockSpec((1, H // 2), lambda b: (0, 0)),
            pl.BlockSpec((1, H // 2), lambda b: (0, 0)),
            pl.BlockSpec((1, 1), lambda b: (0, 0)),
        ],
        out_specs=pl.BlockSpec((BLK, 1), lambda b: (b, 0)),
        out_shape=jax.ShapeDtypeStruct((N, 1), jnp.float32),
    )(m, w1, b1r, w2r, b2r)


def kernel(X, edge_index, W0, b0, Wih, Whh, bih, bhh, W1, b1, W2, b2):
    row = edge_index[0]
    col = edge_index[1]

    rowp = jnp.concatenate(
        [row, jnp.zeros((EPAD - E,), jnp.int32)]).reshape(NS, NCH, CH)
    colp = jnp.concatenate(
        [col, jnp.full((EPAD - E,), 1 << 29, jnp.int32)]).reshape(NS, NCH, CH)
    rc = jnp.concatenate([row, col,
                          jnp.full((DPAD - 2 * E,), TRASH_D, jnp.int32)])
    rc_t = rc.reshape(NC, NS, NCH_D, CH)

    zb = jnp.zeros((ZCH, H), jnp.float32)
    ones1 = jnp.ones((CH, DEG_W), jnp.float32)
    zeros_d = jnp.zeros((DZR, DEG_W), jnp.float32)

    xp = jnp.pad(X, ((0, 0), (0, 8 - X.shape[1])))
    w0p = jnp.pad(W0, ((0, 8 - W0.shape[0]), (0, 0)))
    b0r = b0.reshape(1, H)
    wt_i = Wih.T
    wt_h = Whh.T
    ws = (wt_i[:, 0:H], wt_i[:, H:2 * H], wt_i[:, 2 * H:3 * H],
          wt_h[:, 0:H], wt_h[:, H:2 * H], wt_h[:, 2 * H:3 * H])
    bs = ((bih[0:H] + bhh[0:H]).reshape(1, H),
          (bih[H:2 * H] + bhh[H:2 * H]).reshape(1, H),
          bih[2 * H:3 * H].reshape(1, H),
          bhh[2 * H:3 * H].reshape(1, H))
    b1r = b1.reshape(1, H // 2)
    w2r = W2.T
    b2r = b2.reshape(1, 1)

    dparts = _deg(rc_t, ones1, zeros_d)

    h, hs, dinv = _tc_init(xp, w0p, b0r, dparts)

    m = jnp.full((N, H), -jnp.inf, jnp.float32)
    for _ in range(DEPTH - 1):
        agg = _prop(hs, rowp, colp, zb)
        h, hs, m = _tc_gru(agg, h, dinv, m, ws, bs)

    out = _tc_final(m, W1, b1r, w2r, b2r)
    return out.reshape(N)
